# Initial kernel scaffold; baseline (speedup 1.0000x reference)
import jax
import jax.numpy as jnp
from jax.experimental import pallas as pl


def kernel(x, edge_index, W1, att_src1, att_dst1, b1, W2, att_src2, att_dst2, b2):
    raise NotImplementedError("write your pallas kernel here")


# speedup vs baseline: 44.7723x; 44.7723x over previous
import functools

import jax
import jax.numpy as jnp
from jax import lax
from jax.experimental import pallas as pl
from jax.experimental.pallas import tpu as pltpu
from jax.experimental.pallas import tpu_sc as plsc

_N = 10000
_E = 320000
_D = 128

_NC = 2
_NS = 16
_LN = 16
_NW = _NC * _NS
_EPW = _E // _NW
_K = 80
_NCHUNK = _EPW // _K
_NP = 10240
_RPT = _NP // _NS
_ZR = 128


def _dense1_body(x_ref, w_ref, asr_ref, adr_ref, h_ref, as_ref, ad_ref, m_ref):
    h = jnp.dot(x_ref[...], w_ref[...], preferred_element_type=jnp.float32)
    h_ref[...] = h
    a_s = jnp.sum(h * asr_ref[...], axis=1, keepdims=True)
    a_d = jnp.sum(h * adr_ref[...], axis=1, keepdims=True)
    as_ref[...] = a_s
    ad_ref[...] = a_d
    m = jnp.max(a_s) + jnp.max(a_d)
    m = jnp.where(m >= 0, m, 0.2 * m)
    m_ref[...] = jnp.zeros((1, 1), jnp.float32) + m


def _dense2_body(p_ref, b1_ref, w2_ref, asr_ref, adr_ref,
                 h_ref, as_ref, ad_ref, m_ref):
    num = p_ref[0, :_N] + p_ref[1, :_N]
    den = num[:, 8:9]
    z = jnp.maximum(num[:, :8] / (den + 1e-16) + b1_ref[...], 0.0)
    h2 = jnp.dot(z, w2_ref[...], preferred_element_type=jnp.float32)
    h_ref[...] = h2
    a_s = jnp.sum(h2 * asr_ref[...], axis=1, keepdims=True)
    a_d = jnp.sum(h2 * adr_ref[...], axis=1, keepdims=True)
    as_ref[...] = a_s
    ad_ref[...] = a_d
    m = jnp.max(a_s) + jnp.max(a_d)
    m = jnp.where(m >= 0, m, 0.2 * m)
    m_ref[...] = jnp.zeros((1, 1), jnp.float32) + m


def _final_body(p_ref, b2_ref, out_ref):
    num = p_ref[0, :_N] + p_ref[1, :_N]
    den = num[:, 10:11]
    o = num[:, :10] / (den + 1e-16) + b2_ref[...]
    mx = jnp.max(o, axis=1, keepdims=True)
    lse = jnp.log(jnp.sum(jnp.exp(o - mx), axis=1, keepdims=True)) + mx
    out_ref[...] = o - lse


def _dense1(x, w1, att_s, att_d):
    return pl.pallas_call(
        _dense1_body,
        out_shape=(
            jax.ShapeDtypeStruct((_N, 8), jnp.float32),
            jax.ShapeDtypeStruct((_N, 1), jnp.float32),
            jax.ShapeDtypeStruct((_N, 1), jnp.float32),
            jax.ShapeDtypeStruct((1, 1), jnp.float32),
        ),
    )(x, w1, att_s.reshape(1, 8), att_d.reshape(1, 8))


def _dense2(p1, b1, w2, att_s, att_d):
    return pl.pallas_call(
        _dense2_body,
        out_shape=(
            jax.ShapeDtypeStruct((_N, 10), jnp.float32),
            jax.ShapeDtypeStruct((_N, 1), jnp.float32),
            jax.ShapeDtypeStruct((_N, 1), jnp.float32),
            jax.ShapeDtypeStruct((1, 1), jnp.float32),
        ),
    )(p1, b1.reshape(1, 8), w2, att_s.reshape(1, 10), att_d.reshape(1, 10))


def _final(p2, b2):
    return pl.pallas_call(
        _final_body,
        out_shape=jax.ShapeDtypeStruct((_N, 10), jnp.float32),
    )(p2, b2.reshape(1, 10))


def _edge_body(F, ei_hbm, hw_hbm, as_hbm, ad_hbm, m_hbm, out_hbm,
               hw_v, as_v, ad_v, m_v, src_v, dst_v, cbuf, zbuf, shared):
    HW = F // 2
    c = lax.axis_index("c")
    s = lax.axis_index("s")
    w = c * _NS + s

    pltpu.sync_copy(hw_hbm, hw_v)
    pltpu.sync_copy(as_hbm, as_v)
    pltpu.sync_copy(ad_hbm, ad_v)
    pltpu.sync_copy(m_hbm, m_v)

    zero16 = jnp.zeros((_LN,), jnp.float32)

    def zrow(i, carry):
        zbuf[i, :] = zero16
        return carry
    lax.fori_loop(0, _ZR, zrow, 0)

    def crow(i, carry):
        cbuf[i, :] = zero16
        return carry
    lax.fori_loop(0, _K, crow, 0)

    row0 = s * _RPT
    for j in range(_RPT // _ZR):
        pltpu.sync_copy(zbuf, shared.at[pl.ds(row0 + j * _ZR, _ZR)])
    plsc.subcore_barrier()

    mvec = m_v[...]
    col_den = jnp.full((_LN,), F, jnp.int32)
    lanes = lax.iota(jnp.int32, _LN)
    himask = jnp.full((_LN,), -65536, jnp.int32)

    def chunk(ci, carry):
        base = w * _EPW + ci * _K
        pltpu.sync_copy(ei_hbm.at[pl.ds(base, _K)], src_v)
        pltpu.sync_copy(ei_hbm.at[pl.ds(_E + base, _K)], dst_v)
        for v in range(_K // _LN):
            sl = pl.ds(v * _LN, _LN)
            sv = src_v[sl]
            dv = dst_v[sl]
            a = plsc.load_gather(as_v, [sv]) + plsc.load_gather(ad_v, [dv])
            a = jnp.where(a >= 0, a, 0.2 * a)
            ex = jnp.exp(a - mvec)
            rows = lanes + (v * _LN)
            plsc.store_scatter(cbuf, [rows, col_den], ex)
            svw = sv * HW
            for k in range(HW):
                wv = plsc.load_gather(hw_v, [svw + k])
                flo = plsc.bitcast(lax.shift_left(wv, 16), jnp.float32)
                fhi = plsc.bitcast(wv & himask, jnp.float32)
                plsc.store_scatter(
                    cbuf, [rows, jnp.full((_LN,), 2 * k, jnp.int32)], ex * flo)
                plsc.store_scatter(
                    cbuf, [rows, jnp.full((_LN,), 2 * k + 1, jnp.int32)], ex * fhi)
        pltpu.sync_copy(cbuf, shared.at[dst_v], add=True)
        return carry

    lax.fori_loop(0, _NCHUNK, chunk, 0)
    plsc.subcore_barrier()

    pltpu.sync_copy(shared.at[pl.ds(row0, _RPT)],
                    out_hbm.at[c, pl.ds(row0, _RPT)])


@functools.lru_cache(maxsize=None)
def _edge_pass(F):
    mesh = plsc.VectorSubcoreMesh(core_axis_name="c", subcore_axis_name="s",
                                  num_cores=_NC, num_subcores=_NS)
    return pl.kernel(
        functools.partial(_edge_body, F),
        out_type=jax.ShapeDtypeStruct((_NC, _NP, _LN), jnp.float32),
        mesh=mesh,
        compiler_params=pltpu.CompilerParams(needs_layout_passes=False,
                                             use_tc_tiling_on_sc=False),
        scratch_types=[
            pltpu.VMEM((_N * (F // 2),), jnp.int32),
            pltpu.VMEM((_N,), jnp.float32),
            pltpu.VMEM((_N,), jnp.float32),
            pltpu.VMEM((_LN,), jnp.float32),
            pltpu.VMEM((_K,), jnp.int32),
            pltpu.VMEM((_K,), jnp.int32),
            pltpu.VMEM((_K, _LN), jnp.float32),
            pltpu.VMEM((_ZR, _LN), jnp.float32),
            pltpu.VMEM_SHARED((_NP, _LN), jnp.float32),
        ],
    )


def _pack_bf16(h, F):
    hb = h.astype(jnp.bfloat16)
    return jax.lax.bitcast_convert_type(
        hb.reshape(_N, F // 2, 2), jnp.int32).reshape(_N * (F // 2))


def kernel(x, edge_index, W1, att_src1, att_dst1, b1,
           W2, att_src2, att_dst2, b2):
    ei = edge_index.reshape(2 * _E)
    h1, a1s, a1d, m1 = _dense1(x, W1, att_src1, att_dst1)
    m1v = jnp.broadcast_to(m1.reshape(()), (_LN,))
    p1 = _edge_pass(8)(ei, _pack_bf16(h1, 8),
                       a1s.reshape(_N), a1d.reshape(_N), m1v)
    h2, a2s, a2d, m2 = _dense2(p1, b1, W2, att_src2, att_dst2)
    m2v = jnp.broadcast_to(m2.reshape(()), (_LN,))
    p2 = _edge_pass(10)(ei, _pack_bf16(h2, 10),
                        a2s.reshape(_N), a2d.reshape(_N), m2v)
    return _final(p2, b2)


# speedup vs baseline: 88.5236x; 1.9772x over previous
import functools

import jax
import jax.numpy as jnp
from jax import lax
from jax.experimental import pallas as pl
from jax.experimental.pallas import tpu as pltpu
from jax.experimental.pallas import tpu_sc as plsc

_N = 10000
_E = 320000
_D = 128

_NC = 2
_NS = 16
_LN = 16
_NW = _NC * _NS
_EPW = _E // _NW
_K = 80
_NCHUNK = _EPW // _K
_NB = 5
_NP = 10240
_RPT = _NP // _NS
_ZR = 128


def _dense1_body(x_ref, w_ref, asr_ref, adr_ref, h_ref, as_ref, ad_ref, m_ref):
    h = jnp.dot(x_ref[...], w_ref[...], preferred_element_type=jnp.float32)
    h_ref[...] = h
    a_s = jnp.sum(h * asr_ref[...], axis=1, keepdims=True)
    a_d = jnp.sum(h * adr_ref[...], axis=1, keepdims=True)
    as_ref[...] = a_s
    ad_ref[...] = a_d
    m = jnp.max(a_s) + jnp.max(a_d)
    m = jnp.where(m >= 0, m, 0.2 * m)
    m_ref[...] = jnp.zeros((1, 1), jnp.float32) + m


def _dense2_body(p_ref, b1_ref, w2_ref, asr_ref, adr_ref,
                 h_ref, as_ref, ad_ref, m_ref):
    num = p_ref[0, :_N] + p_ref[1, :_N]
    den = num[:, 8:9]
    z = jnp.maximum(num[:, :8] / (den + 1e-16) + b1_ref[...], 0.0)
    h2 = jnp.dot(z, w2_ref[...], preferred_element_type=jnp.float32)
    h_ref[...] = h2
    a_s = jnp.sum(h2 * asr_ref[...], axis=1, keepdims=True)
    a_d = jnp.sum(h2 * adr_ref[...], axis=1, keepdims=True)
    as_ref[...] = a_s
    ad_ref[...] = a_d
    m = jnp.max(a_s) + jnp.max(a_d)
    m = jnp.where(m >= 0, m, 0.2 * m)
    m_ref[...] = jnp.zeros((1, 1), jnp.float32) + m


def _final_body(p_ref, b2_ref, out_ref):
    num = p_ref[0, :_N] + p_ref[1, :_N]
    den = num[:, 10:11]
    o = num[:, :10] / (den + 1e-16) + b2_ref[...]
    mx = jnp.max(o, axis=1, keepdims=True)
    lse = jnp.log(jnp.sum(jnp.exp(o - mx), axis=1, keepdims=True)) + mx
    out_ref[...] = o - lse


def _dense1(x, w1, att_s, att_d):
    return pl.pallas_call(
        _dense1_body,
        out_shape=(
            jax.ShapeDtypeStruct((_N, 8), jnp.float32),
            jax.ShapeDtypeStruct((_N, 1), jnp.float32),
            jax.ShapeDtypeStruct((_N, 1), jnp.float32),
            jax.ShapeDtypeStruct((1, 1), jnp.float32),
        ),
    )(x, w1, att_s.reshape(1, 8), att_d.reshape(1, 8))


def _dense2(p1, b1, w2, att_s, att_d):
    return pl.pallas_call(
        _dense2_body,
        out_shape=(
            jax.ShapeDtypeStruct((_N, 10), jnp.float32),
            jax.ShapeDtypeStruct((_N, 1), jnp.float32),
            jax.ShapeDtypeStruct((_N, 1), jnp.float32),
            jax.ShapeDtypeStruct((1, 1), jnp.float32),
        ),
    )(p1, b1.reshape(1, 8), w2, att_s.reshape(1, 10), att_d.reshape(1, 10))


def _final(p2, b2):
    return pl.pallas_call(
        _final_body,
        out_shape=jax.ShapeDtypeStruct((_N, 10), jnp.float32),
    )(p2, b2.reshape(1, 10))


def _edge_body(F, ei_hbm, hw_hbm, as_hbm, ad_hbm, m_hbm, out_hbm,
               hw_v, as_v, ad_v, m_v, src_v, dst_v, cbuf, zbuf, shared,
               lsem_s, lsem_d, ssem):
    HW = F // 2
    c = lax.axis_index("c")
    s = lax.axis_index("s")
    w = c * _NS + s

    pltpu.sync_copy(hw_hbm, hw_v)
    pltpu.sync_copy(as_hbm, as_v)
    pltpu.sync_copy(ad_hbm, ad_v)
    pltpu.sync_copy(m_hbm, m_v)

    zero16 = jnp.zeros((_LN,), jnp.float32)

    def zrow(i, carry):
        zbuf[i, :] = zero16
        return carry
    lax.fori_loop(0, _ZR, zrow, 0)

    for b in range(_NB):
        def crow(i, carry, b=b):
            cbuf[b, i, :] = zero16
            return carry
        lax.fori_loop(0, _K, crow, 0)

    row0 = s * _RPT
    for j in range(_RPT // _ZR):
        pltpu.sync_copy(zbuf, shared.at[pl.ds(row0 + j * _ZR, _ZR)])
    plsc.subcore_barrier()

    mvec = m_v[...]
    col_den = jnp.full((_LN,), F, jnp.int32)
    cols = [jnp.full((_LN,), f, jnp.int32) for f in range(F)]
    lanes = lax.iota(jnp.int32, _LN)
    himask = jnp.full((_LN,), -65536, jnp.int32)

    def issue_load(b, ci):
        base = w * _EPW + ci * _K
        pltpu.async_copy(ei_hbm.at[pl.ds(base, _K)], src_v.at[b], lsem_s.at[b])
        pltpu.async_copy(ei_hbm.at[pl.ds(_E + base, _K)], dst_v.at[b],
                         lsem_d.at[b])

    def wait_load(b):
        pltpu.make_async_copy(ei_hbm.at[pl.ds(0, _K)], src_v.at[b],
                              lsem_s.at[b]).wait()
        pltpu.make_async_copy(ei_hbm.at[pl.ds(0, _K)], dst_v.at[b],
                              lsem_d.at[b]).wait()

    def issue_scat(b):
        pltpu.async_copy(cbuf.at[b], shared.at[dst_v.at[b]], ssem.at[b],
                         add=True)

    def wait_scat(b):
        pltpu.make_async_copy(cbuf.at[b], shared.at[dst_v.at[b]],
                              ssem.at[b]).wait()

    def compute(b):
        for v in range(_K // _LN):
            sl = pl.ds(v * _LN, _LN)
            sv = src_v[b, sl]
            dv = dst_v[b, sl]
            a = plsc.load_gather(as_v, [sv]) + plsc.load_gather(ad_v, [dv])
            a = jnp.where(a >= 0, a, 0.2 * a)
            ex = jnp.exp(a - mvec)
            rows = lanes + (v * _LN)
            cb = cbuf.at[b]
            plsc.store_scatter(cb, [rows, col_den], ex)
            svw = sv * HW
            for k in range(HW):
                wv = plsc.load_gather(hw_v, [svw + k])
                flo = plsc.bitcast(lax.shift_left(wv, 16), jnp.float32)
                fhi = plsc.bitcast(wv & himask, jnp.float32)
                plsc.store_scatter(cb, [rows, cols[2 * k]], ex * flo)
                plsc.store_scatter(cb, [rows, cols[2 * k + 1]], ex * fhi)

    for b in range(_NB - 1):
        issue_load(b, jnp.int32(b))

    def step(ci, b, first):
        wait_load(b)
        compute(b)
        issue_scat(b)
        pj = (b - 1) % _NB
        if not first:
            wait_scat(pj)
        nxt = ci + (_NB - 1)

        @pl.when(nxt < _NCHUNK)
        def _():
            issue_load(pj, nxt)

    for j in range(_NB):
        step(jnp.int32(j), j, first=(j == 0))

    def group(g, carry):
        for j in range(_NB):
            step(g * _NB + j, j, first=False)
        return carry

    lax.fori_loop(1, _NCHUNK // _NB, group, 0)
    wait_scat((_NCHUNK - 1) % _NB)
    plsc.subcore_barrier()

    pltpu.sync_copy(shared.at[pl.ds(row0, _RPT)],
                    out_hbm.at[c, pl.ds(row0, _RPT)])


@functools.lru_cache(maxsize=None)
def _edge_pass(F):
    mesh = plsc.VectorSubcoreMesh(core_axis_name="c", subcore_axis_name="s",
                                  num_cores=_NC, num_subcores=_NS)
    return pl.kernel(
        functools.partial(_edge_body, F),
        out_type=jax.ShapeDtypeStruct((_NC, _NP, _LN), jnp.float32),
        mesh=mesh,
        compiler_params=pltpu.CompilerParams(needs_layout_passes=False,
                                             use_tc_tiling_on_sc=False),
        scratch_types=[
            pltpu.VMEM((_N * (F // 2),), jnp.int32),
            pltpu.VMEM((_N,), jnp.float32),
            pltpu.VMEM((_N,), jnp.float32),
            pltpu.VMEM((_LN,), jnp.float32),
            pltpu.VMEM((_NB, _K), jnp.int32),
            pltpu.VMEM((_NB, _K), jnp.int32),
            pltpu.VMEM((_NB, _K, _LN), jnp.float32),
            pltpu.VMEM((_ZR, _LN), jnp.float32),
            pltpu.VMEM_SHARED((_NP, _LN), jnp.float32),
            pltpu.SemaphoreType.DMA((_NB,)),
            pltpu.SemaphoreType.DMA((_NB,)),
            pltpu.SemaphoreType.DMA((_NB,)),
        ],
    )


def _pack_bf16(h, F):
    hb = h.astype(jnp.bfloat16)
    return jax.lax.bitcast_convert_type(
        hb.reshape(_N, F // 2, 2), jnp.int32).reshape(_N * (F // 2))


def kernel(x, edge_index, W1, att_src1, att_dst1, b1,
           W2, att_src2, att_dst2, b2):
    ei = edge_index.reshape(2 * _E)
    h1, a1s, a1d, m1 = _dense1(x, W1, att_src1, att_dst1)
    m1v = jnp.broadcast_to(m1.reshape(()), (_LN,))
    p1 = _edge_pass(8)(ei, _pack_bf16(h1, 8),
                       a1s.reshape(_N), a1d.reshape(_N), m1v)
    h2, a2s, a2d, m2 = _dense2(p1, b1, W2, att_src2, att_dst2)
    m2v = jnp.broadcast_to(m2.reshape(()), (_LN,))
    p2 = _edge_pass(10)(ei, _pack_bf16(h2, 10),
                        a2s.reshape(_N), a2d.reshape(_N), m2v)
    return _final(p2, b2)


# speedup vs baseline: 92.2847x; 1.0425x over previous
import functools

import jax
import jax.numpy as jnp
from jax import lax
from jax.experimental import pallas as pl
from jax.experimental.pallas import tpu as pltpu
from jax.experimental.pallas import tpu_sc as plsc

_N = 10000
_E = 320000
_D = 128

_NC = 2
_NS = 16
_LN = 16
_NW = _NC * _NS
_EPW = _E // _NW
_K = 80
_NCHUNK = _EPW // _K
_NB = 5
_NP = 10240
_RPT = _NP // _NS
_ZR = 128


def _dense1_body(x_ref, w_ref, asr_ref, adr_ref, h_ref, as_ref, ad_ref, m_ref):
    h = jnp.dot(x_ref[...], w_ref[...], preferred_element_type=jnp.float32)
    h_ref[...] = h
    a_s = jnp.sum(h * asr_ref[...], axis=1, keepdims=True)
    a_d = jnp.sum(h * adr_ref[...], axis=1, keepdims=True)
    as_ref[...] = a_s
    ad_ref[...] = a_d
    m = jnp.max(a_s) + jnp.max(a_d)
    m = jnp.where(m >= 0, m, 0.2 * m)
    m_ref[...] = jnp.zeros((1, 1), jnp.float32) + m


def _dense2_body(p_ref, b1_ref, w2_ref, asr_ref, adr_ref,
                 h_ref, as_ref, ad_ref, m_ref):
    num = p_ref[0, :_N] + p_ref[1, :_N]
    den = num[:, 8:9]
    z = jnp.maximum(num[:, :8] / (den + 1e-16) + b1_ref[...], 0.0)
    h2 = jnp.dot(z, w2_ref[...], preferred_element_type=jnp.float32)
    h_ref[...] = h2
    a_s = jnp.sum(h2 * asr_ref[...], axis=1, keepdims=True)
    a_d = jnp.sum(h2 * adr_ref[...], axis=1, keepdims=True)
    as_ref[...] = a_s
    ad_ref[...] = a_d
    m = jnp.max(a_s) + jnp.max(a_d)
    m = jnp.where(m >= 0, m, 0.2 * m)
    m_ref[...] = jnp.zeros((1, 1), jnp.float32) + m


def _final_body(p_ref, b2_ref, out_ref):
    num = p_ref[0, :_N] + p_ref[1, :_N]
    den = num[:, 10:11]
    o = num[:, :10] / (den + 1e-16) + b2_ref[...]
    mx = jnp.max(o, axis=1, keepdims=True)
    lse = jnp.log(jnp.sum(jnp.exp(o - mx), axis=1, keepdims=True)) + mx
    out_ref[...] = o - lse


def _dense1(x, w1, att_s, att_d):
    return pl.pallas_call(
        _dense1_body,
        out_shape=(
            jax.ShapeDtypeStruct((_N, 8), jnp.float32),
            jax.ShapeDtypeStruct((_N, 1), jnp.float32),
            jax.ShapeDtypeStruct((_N, 1), jnp.float32),
            jax.ShapeDtypeStruct((1, 1), jnp.float32),
        ),
    )(x, w1, att_s.reshape(1, 8), att_d.reshape(1, 8))


def _dense2(p1, b1, w2, att_s, att_d):
    return pl.pallas_call(
        _dense2_body,
        out_shape=(
            jax.ShapeDtypeStruct((_N, 10), jnp.float32),
            jax.ShapeDtypeStruct((_N, 1), jnp.float32),
            jax.ShapeDtypeStruct((_N, 1), jnp.float32),
            jax.ShapeDtypeStruct((1, 1), jnp.float32),
        ),
    )(p1, b1.reshape(1, 8), w2, att_s.reshape(1, 10), att_d.reshape(1, 10))


def _final(p2, b2):
    return pl.pallas_call(
        _final_body,
        out_shape=jax.ShapeDtypeStruct((_N, 10), jnp.float32),
    )(p2, b2.reshape(1, 10))


def _edge_body(F, ei_hbm, hw_hbm, as_hbm, ad_hbm, m_hbm, out_hbm,
               hw_v, as_v, ad_v, m_v, src_v, dst_v, cbuf, zbuf, shared,
               lsem_s, lsem_d, ssem, tsem, zsem):
    HW = F // 2
    c = lax.axis_index("c")
    s = lax.axis_index("s")
    w = c * _NS + s

    pltpu.async_copy(hw_hbm, hw_v, tsem.at[0])
    pltpu.async_copy(as_hbm, as_v, tsem.at[1])
    pltpu.async_copy(ad_hbm, ad_v, tsem.at[2])
    pltpu.async_copy(m_hbm, m_v, tsem.at[3])

    zero16 = jnp.zeros((_LN,), jnp.float32)

    def zrow(i, carry):
        zbuf[i, :] = zero16
        return carry
    lax.fori_loop(0, _ZR, zrow, 0)

    row0 = s * _RPT
    for j in range(_RPT // _ZR):
        pltpu.async_copy(zbuf, shared.at[pl.ds(row0 + j * _ZR, _ZR)], zsem)
    for j in range(_RPT // _ZR):
        pltpu.make_async_copy(zbuf, shared.at[pl.ds(row0, _ZR)], zsem).wait()
    pltpu.make_async_copy(hw_hbm, hw_v, tsem.at[0]).wait()
    pltpu.make_async_copy(as_hbm, as_v, tsem.at[1]).wait()
    pltpu.make_async_copy(ad_hbm, ad_v, tsem.at[2]).wait()
    pltpu.make_async_copy(m_hbm, m_v, tsem.at[3]).wait()
    plsc.subcore_barrier()

    mvec = m_v[...]
    col_den = jnp.full((_LN,), F, jnp.int32)
    cols = [jnp.full((_LN,), f, jnp.int32) for f in range(F)]
    lanes = lax.iota(jnp.int32, _LN)
    himask = jnp.full((_LN,), -65536, jnp.int32)

    def issue_load(b, ci):
        base = w * _EPW + ci * _K
        pltpu.async_copy(ei_hbm.at[pl.ds(base, _K)], src_v.at[b], lsem_s.at[b])
        pltpu.async_copy(ei_hbm.at[pl.ds(_E + base, _K)], dst_v.at[b],
                         lsem_d.at[b])

    def wait_load(b):
        pltpu.make_async_copy(ei_hbm.at[pl.ds(0, _K)], src_v.at[b],
                              lsem_s.at[b]).wait()
        pltpu.make_async_copy(ei_hbm.at[pl.ds(0, _K)], dst_v.at[b],
                              lsem_d.at[b]).wait()

    def issue_scat(b):
        pltpu.async_copy(cbuf.at[b], shared.at[dst_v.at[b]], ssem.at[b],
                         add=True)

    def wait_scat(b):
        pltpu.make_async_copy(cbuf.at[b], shared.at[dst_v.at[b]],
                              ssem.at[b]).wait()

    def compute(b):
        for v in range(_K // _LN):
            sl = pl.ds(v * _LN, _LN)
            sv = src_v[b, sl]
            dv = dst_v[b, sl]
            a = plsc.load_gather(as_v, [sv]) + plsc.load_gather(ad_v, [dv])
            a = jnp.maximum(a, 0.2 * a)
            ex = jnp.exp(a - mvec)
            rows = lanes + (v * _LN)
            cb = cbuf.at[b]
            plsc.store_scatter(cb, [rows, col_den], ex)
            svw = sv * HW
            for k in range(HW):
                wv = plsc.load_gather(hw_v, [svw + k])
                flo = plsc.bitcast(lax.shift_left(wv, 16), jnp.float32)
                fhi = plsc.bitcast(wv & himask, jnp.float32)
                plsc.store_scatter(cb, [rows, cols[2 * k]], ex * flo)
                plsc.store_scatter(cb, [rows, cols[2 * k + 1]], ex * fhi)

    for b in range(_NB - 2):
        issue_load(b, jnp.int32(b))

    def step(ci, b, first):
        wait_load(b)
        compute(b)
        issue_scat(b)
        pj = (b - 2) % _NB
        if not first:
            wait_scat(pj)
        nxt = ci + (_NB - 2)

        @pl.when(nxt < _NCHUNK)
        def _():
            issue_load(pj, nxt)

    for j in range(_NB):
        step(jnp.int32(j), j, first=(j <= 1))

    def group(g, carry):
        for j in range(_NB):
            step(g * _NB + j, j, first=False)
        return carry

    lax.fori_loop(1, _NCHUNK // _NB, group, 0)
    wait_scat((_NCHUNK - 2) % _NB)
    wait_scat((_NCHUNK - 1) % _NB)
    plsc.subcore_barrier()

    pltpu.sync_copy(shared.at[pl.ds(row0, _RPT)],
                    out_hbm.at[c, pl.ds(row0, _RPT)])


@functools.lru_cache(maxsize=None)
def _edge_pass(F):
    mesh = plsc.VectorSubcoreMesh(core_axis_name="c", subcore_axis_name="s",
                                  num_cores=_NC, num_subcores=_NS)
    return pl.kernel(
        functools.partial(_edge_body, F),
        out_type=jax.ShapeDtypeStruct((_NC, _NP, _LN), jnp.float32),
        mesh=mesh,
        compiler_params=pltpu.CompilerParams(needs_layout_passes=False,
                                             use_tc_tiling_on_sc=False),
        scratch_types=[
            pltpu.VMEM((_N * (F // 2),), jnp.int32),
            pltpu.VMEM((_N,), jnp.float32),
            pltpu.VMEM((_N,), jnp.float32),
            pltpu.VMEM((_LN,), jnp.float32),
            pltpu.VMEM((_NB, _K), jnp.int32),
            pltpu.VMEM((_NB, _K), jnp.int32),
            pltpu.VMEM((_NB, _K, _LN), jnp.float32),
            pltpu.VMEM((_ZR, _LN), jnp.float32),
            pltpu.VMEM_SHARED((_NP, _LN), jnp.float32),
            pltpu.SemaphoreType.DMA((_NB,)),
            pltpu.SemaphoreType.DMA((_NB,)),
            pltpu.SemaphoreType.DMA((_NB,)),
            pltpu.SemaphoreType.DMA((4,)),
            pltpu.SemaphoreType.DMA,
        ],
    )


def _pack_bf16(h, F):
    hb = h.astype(jnp.bfloat16)
    return jax.lax.bitcast_convert_type(
        hb.reshape(_N, F // 2, 2), jnp.int32).reshape(_N * (F // 2))


def kernel(x, edge_index, W1, att_src1, att_dst1, b1,
           W2, att_src2, att_dst2, b2):
    ei = edge_index.reshape(2 * _E)
    h1, a1s, a1d, m1 = _dense1(x, W1, att_src1, att_dst1)
    m1v = jnp.broadcast_to(m1.reshape(()), (_LN,))
    p1 = _edge_pass(8)(ei, _pack_bf16(h1, 8),
                       a1s.reshape(_N), a1d.reshape(_N), m1v)
    h2, a2s, a2d, m2 = _dense2(p1, b1, W2, att_src2, att_dst2)
    m2v = jnp.broadcast_to(m2.reshape(()), (_LN,))
    p2 = _edge_pass(10)(ei, _pack_bf16(h2, 10),
                        a2s.reshape(_N), a2d.reshape(_N), m2v)
    return _final(p2, b2)


# speedup vs baseline: 100.2249x; 1.0860x over previous
import functools

import jax
import jax.numpy as jnp
from jax import lax
from jax.experimental import pallas as pl
from jax.experimental.pallas import tpu as pltpu
from jax.experimental.pallas import tpu_sc as plsc

_N = 10000
_E = 320000
_D = 128

_NC = 2
_NS = 16
_LN = 16
_NW = _NC * _NS
_EPW = _E // _NW
_K = 80
_NCHUNK = _EPW // _K
_NB = 5
_NP = 10240
_RPT = _NP // _NS
_ZR = 128


def _dense1_body(x_ref, w_ref, asr_ref, adr_ref, h_ref, as_ref, ad_ref, m_ref):
    h = jnp.dot(x_ref[...], w_ref[...], preferred_element_type=jnp.float32)
    h_ref[...] = h
    a_s = jnp.dot(h, asr_ref[...], preferred_element_type=jnp.float32)
    a_d = jnp.dot(h, adr_ref[...], preferred_element_type=jnp.float32)
    as_ref[...] = a_s
    ad_ref[...] = a_d
    m = jnp.max(a_s) + jnp.max(a_d)
    m = jnp.where(m >= 0, m, 0.2 * m)
    m_ref[...] = jnp.zeros((1, 1), jnp.float32) + m


def _dense2_body(p_ref, b1_ref, w2_ref, asr_ref, adr_ref,
                 h_ref, as_ref, ad_ref, m_ref):
    num = p_ref[0, :_N] + p_ref[1, :_N]
    den = num[:, 8:9]
    z = jnp.maximum(num[:, :8] / (den + 1e-16) + b1_ref[...], 0.0)
    h2 = jnp.dot(z, w2_ref[...], preferred_element_type=jnp.float32)
    h_ref[...] = h2
    a_s = jnp.dot(h2, asr_ref[...], preferred_element_type=jnp.float32)
    a_d = jnp.dot(h2, adr_ref[...], preferred_element_type=jnp.float32)
    as_ref[...] = a_s
    ad_ref[...] = a_d
    m = jnp.max(a_s) + jnp.max(a_d)
    m = jnp.where(m >= 0, m, 0.2 * m)
    m_ref[...] = jnp.zeros((1, 1), jnp.float32) + m


def _final_body(p_ref, b2_ref, out_ref):
    num = p_ref[0, :_N] + p_ref[1, :_N]
    den = num[:, 10:11]
    o = num[:, :10] / (den + 1e-16) + b2_ref[...]
    mx = jnp.max(o, axis=1, keepdims=True)
    e = jnp.exp(o - mx)
    ones = jnp.ones((10, 1), jnp.float32)
    lse = jnp.log(jnp.dot(e, ones, preferred_element_type=jnp.float32)) + mx
    out_ref[...] = o - lse


def _dense1(x, w1, att_s, att_d):
    return pl.pallas_call(
        _dense1_body,
        out_shape=(
            jax.ShapeDtypeStruct((_N, 8), jnp.float32),
            jax.ShapeDtypeStruct((_N, 1), jnp.float32),
            jax.ShapeDtypeStruct((_N, 1), jnp.float32),
            jax.ShapeDtypeStruct((1, 1), jnp.float32),
        ),
    )(x, w1, att_s.reshape(8, 1), att_d.reshape(8, 1))


def _dense2(p1, b1, w2, att_s, att_d):
    return pl.pallas_call(
        _dense2_body,
        out_shape=(
            jax.ShapeDtypeStruct((_N, 10), jnp.float32),
            jax.ShapeDtypeStruct((_N, 1), jnp.float32),
            jax.ShapeDtypeStruct((_N, 1), jnp.float32),
            jax.ShapeDtypeStruct((1, 1), jnp.float32),
        ),
    )(p1, b1.reshape(1, 8), w2, att_s.reshape(10, 1), att_d.reshape(10, 1))


def _final(p2, b2):
    return pl.pallas_call(
        _final_body,
        out_shape=jax.ShapeDtypeStruct((_N, 10), jnp.float32),
    )(p2, b2.reshape(1, 10))


def _edge_body(F, ei_hbm, hw_hbm, as_hbm, ad_hbm, m_hbm, out_hbm,
               hw_v, as_v, ad_v, m_v, src_v, dst_v, cbuf, zbuf, shared,
               lsem_s, lsem_d, ssem, tsem, zsem):
    HW = F // 2
    c = lax.axis_index("c")
    s = lax.axis_index("s")
    w = c * _NS + s

    pltpu.async_copy(hw_hbm, hw_v, tsem.at[0])
    pltpu.async_copy(as_hbm, as_v, tsem.at[1])
    pltpu.async_copy(ad_hbm, ad_v, tsem.at[2])
    pltpu.async_copy(m_hbm, m_v, tsem.at[3])

    zero16 = jnp.zeros((_LN,), jnp.float32)

    def zrow(i, carry):
        zbuf[i, :] = zero16
        return carry
    lax.fori_loop(0, _ZR, zrow, 0)

    row0 = s * _RPT
    for j in range(_RPT // _ZR):
        pltpu.async_copy(zbuf, shared.at[pl.ds(row0 + j * _ZR, _ZR)], zsem)
    for j in range(_RPT // _ZR):
        pltpu.make_async_copy(zbuf, shared.at[pl.ds(row0, _ZR)], zsem).wait()
    pltpu.make_async_copy(hw_hbm, hw_v, tsem.at[0]).wait()
    pltpu.make_async_copy(as_hbm, as_v, tsem.at[1]).wait()
    pltpu.make_async_copy(ad_hbm, ad_v, tsem.at[2]).wait()
    pltpu.make_async_copy(m_hbm, m_v, tsem.at[3]).wait()
    plsc.subcore_barrier()

    mvec = m_v[...]
    col_den = jnp.full((_LN,), F, jnp.int32)
    cols = [jnp.full((_LN,), f, jnp.int32) for f in range(F)]
    lanes = lax.iota(jnp.int32, _LN)
    himask = jnp.full((_LN,), -65536, jnp.int32)

    def issue_load(b, ci):
        base = w * _EPW + ci * _K
        pltpu.async_copy(ei_hbm.at[pl.ds(base, _K)], src_v.at[b], lsem_s.at[b])
        pltpu.async_copy(ei_hbm.at[pl.ds(_E + base, _K)], dst_v.at[b],
                         lsem_d.at[b])

    def wait_load(b):
        pltpu.make_async_copy(ei_hbm.at[pl.ds(0, _K)], src_v.at[b],
                              lsem_s.at[b]).wait()
        pltpu.make_async_copy(ei_hbm.at[pl.ds(0, _K)], dst_v.at[b],
                              lsem_d.at[b]).wait()

    def issue_scat(b):
        pltpu.async_copy(cbuf.at[b], shared.at[dst_v.at[b]], ssem.at[b],
                         add=True)

    def wait_scat(b):
        pltpu.make_async_copy(cbuf.at[b], shared.at[dst_v.at[b]],
                              ssem.at[b]).wait()

    def compute(b):
        @plsc.parallel_loop(0, _K // _LN, 1, unroll=_K // _LN)
        def _(v):
            sl = pl.ds(v * _LN, _LN)
            sv = src_v[b, sl]
            dv = dst_v[b, sl]
            a = plsc.load_gather(as_v, [sv]) + plsc.load_gather(ad_v, [dv])
            a = jnp.maximum(a, 0.2 * a)
            ex = jnp.exp(a - mvec)
            rows = lanes + (v * _LN)
            cb = cbuf.at[b]
            plsc.store_scatter(cb, [rows, col_den], ex)
            svw = sv * HW
            for k in range(HW):
                wv = plsc.load_gather(hw_v, [svw + k])
                flo = plsc.bitcast(lax.shift_left(wv, 16), jnp.float32)
                fhi = plsc.bitcast(wv & himask, jnp.float32)
                plsc.store_scatter(cb, [rows, cols[2 * k]], ex * flo)
                plsc.store_scatter(cb, [rows, cols[2 * k + 1]], ex * fhi)

    for b in range(_NB - 2):
        issue_load(b, jnp.int32(b))

    def step(ci, b, first):
        wait_load(b)
        compute(b)
        issue_scat(b)
        pj = (b - 2) % _NB
        if not first:
            wait_scat(pj)
        nxt = ci + (_NB - 2)

        @pl.when(nxt < _NCHUNK)
        def _():
            issue_load(pj, nxt)

    for j in range(_NB):
        step(jnp.int32(j), j, first=(j <= 1))

    def group(g, carry):
        for j in range(_NB):
            step(g * _NB + j, j, first=False)
        return carry

    lax.fori_loop(1, _NCHUNK // _NB, group, 0)
    wait_scat((_NCHUNK - 2) % _NB)
    wait_scat((_NCHUNK - 1) % _NB)
    plsc.subcore_barrier()

    pltpu.sync_copy(shared.at[pl.ds(row0, _RPT)],
                    out_hbm.at[c, pl.ds(row0, _RPT)])


@functools.lru_cache(maxsize=None)
def _edge_pass(F):
    mesh = plsc.VectorSubcoreMesh(core_axis_name="c", subcore_axis_name="s",
                                  num_cores=_NC, num_subcores=_NS)
    return pl.kernel(
        functools.partial(_edge_body, F),
        out_type=jax.ShapeDtypeStruct((_NC, _NP, _LN), jnp.float32),
        mesh=mesh,
        compiler_params=pltpu.CompilerParams(needs_layout_passes=False,
                                             use_tc_tiling_on_sc=False),
        scratch_types=[
            pltpu.VMEM((_N * (F // 2),), jnp.int32),
            pltpu.VMEM((_N,), jnp.float32),
            pltpu.VMEM((_N,), jnp.float32),
            pltpu.VMEM((_LN,), jnp.float32),
            pltpu.VMEM((_NB, _K), jnp.int32),
            pltpu.VMEM((_NB, _K), jnp.int32),
            pltpu.VMEM((_NB, _K, _LN), jnp.float32),
            pltpu.VMEM((_ZR, _LN), jnp.float32),
            pltpu.VMEM_SHARED((_NP, _LN), jnp.float32),
            pltpu.SemaphoreType.DMA((_NB,)),
            pltpu.SemaphoreType.DMA((_NB,)),
            pltpu.SemaphoreType.DMA((_NB,)),
            pltpu.SemaphoreType.DMA((4,)),
            pltpu.SemaphoreType.DMA,
        ],
    )


def _pack_bf16(h, F):
    hb = h.astype(jnp.bfloat16)
    return jax.lax.bitcast_convert_type(
        hb.reshape(_N, F // 2, 2), jnp.int32).reshape(_N * (F // 2))


def kernel(x, edge_index, W1, att_src1, att_dst1, b1,
           W2, att_src2, att_dst2, b2):
    ei = edge_index.reshape(2 * _E)
    h1, a1s, a1d, m1 = _dense1(x, W1, att_src1, att_dst1)
    m1v = jnp.broadcast_to(m1.reshape(()), (_LN,))
    p1 = _edge_pass(8)(ei, _pack_bf16(h1, 8),
                       a1s.reshape(_N), a1d.reshape(_N), m1v)
    h2, a2s, a2d, m2 = _dense2(p1, b1, W2, att_src2, att_dst2)
    m2v = jnp.broadcast_to(m2.reshape(()), (_LN,))
    p2 = _edge_pass(10)(ei, _pack_bf16(h2, 10),
                        a2s.reshape(_N), a2d.reshape(_N), m2v)
    return _final(p2, b2)
